# Initial kernel scaffold; baseline (speedup 1.0000x reference)
import jax
import jax.numpy as jnp
from jax.experimental import pallas as pl


def kernel(query, key, value):
    raise NotImplementedError("write your pallas kernel here")


# speedup vs baseline: 2.3762x; 2.3762x over previous
import functools

import jax
import jax.numpy as jnp
from jax.experimental import pallas as pl
from jax.experimental.pallas import tpu as pltpu

_BQ = 512
_NEG = -1e30


def _attn_block_kernel(q_ref, kp_ref, kd_ref, vp_ref, vd_ref, o_ref, *, scale):
    i = pl.program_id(1)
    q = q_ref[0, 0] * scale
    kd = kd_ref[0, 0]
    kp = kp_ref[0, 0]

    s_d = jax.lax.dot_general(q, kd, (((1,), (1,)), ((), ())),
                              preferred_element_type=jnp.float32)
    s_p = jax.lax.dot_general(q, kp, (((1,), (1,)), ((), ())),
                              preferred_element_type=jnp.float32)

    row = jax.lax.broadcasted_iota(jnp.int32, (_BQ, _BQ), 0)
    col = jax.lax.broadcasted_iota(jnp.int32, (_BQ, _BQ), 1)
    s_d = jnp.where(row >= col, s_d, _NEG)
    prev_valid = (row < col) & (i > 0)
    s_p = jnp.where(prev_valid, s_p, _NEG)

    m = jnp.maximum(jnp.max(s_d, axis=-1, keepdims=True),
                    jnp.max(s_p, axis=-1, keepdims=True))
    p_d = jnp.exp(s_d - m)
    p_p = jnp.exp(s_p - m)
    l = jnp.sum(p_d, axis=-1, keepdims=True) + jnp.sum(p_p, axis=-1, keepdims=True)

    acc = jax.lax.dot_general(p_d, vd_ref[0, 0], (((1,), (0,)), ((), ())),
                              preferred_element_type=jnp.float32)
    acc += jax.lax.dot_general(p_p, vp_ref[0, 0], (((1,), (0,)), ((), ())),
                               preferred_element_type=jnp.float32)
    o_ref[0, 0] = acc / l


@jax.jit
def kernel(query, key, value):
    b, h, s, d = query.shape
    scale = 1.0 / (d ** 0.5)
    nq = s // _BQ

    def qo_map(hh, ii):
        return (0, hh, ii, 0)

    def prev_map(hh, ii):
        return (0, hh, jnp.maximum(ii - 1, 0), 0)

    blk = (1, 1, _BQ, d)
    out = pl.pallas_call(
        functools.partial(_attn_block_kernel, scale=scale),
        grid=(h, nq),
        in_specs=[
            pl.BlockSpec(blk, qo_map),
            pl.BlockSpec(blk, prev_map),
            pl.BlockSpec(blk, qo_map),
            pl.BlockSpec(blk, prev_map),
            pl.BlockSpec(blk, qo_map),
        ],
        out_specs=pl.BlockSpec(blk, qo_map),
        out_shape=jax.ShapeDtypeStruct((b, h, s, d), jnp.float32),
    )(query, key, key, value, value)
    return out
